# Initial kernel scaffold; baseline (speedup 1.0000x reference)
import jax
import jax.numpy as jnp
from jax.experimental import pallas as pl


def kernel(batch_size, biggraph, clause_feat, var_feat, graph_pooler, params):
    raise NotImplementedError("write your pallas kernel here")


# speedup vs baseline: 2.2139x; 2.2139x over previous
import jax
import jax.numpy as jnp
from jax.experimental import pallas as pl
from jax.experimental.pallas import tpu as pltpu

NUM_LAYERS = 10
HIDDEN = 16
OUT_DIM = 2
MAXCLAUSE = 1000
MAXVAR = 2000
BATCH = 4
BN_EPS = 1e-5

C = MAXCLAUSE * BATCH
V = MAXVAR * BATCH
N_TOT = 12000.0
BC = 800
NI = C // BC

_F32 = jnp.float32
_BF16 = jnp.bfloat16


def _bn_ref(xc, xv, g, b):
    s = jnp.sum(xc, axis=1, keepdims=True) + jnp.sum(xv, axis=1, keepdims=True)
    m = s / N_TOT
    dc = xc - m
    dv = xv - m
    q = jnp.sum(dc * dc, axis=1, keepdims=True) + jnp.sum(dv * dv, axis=1, keepdims=True)
    v = q / N_TOT
    sd = jnp.sqrt(v + BN_EPS)
    return (dc / sd) * g + b, (dv / sd) * g + b


def _gnn_kernel(b_ref, h0tc_ref, h0tv_ref, w1_ref, w2_ref, b1_ref, g1_ref,
                be1_ref, b2_ref, bng_ref, bnb_ref, eps_ref, w10_ref, out_ref,
                htc, htv, hcbf, hvbf, yc, ytv, dgc, dgv):
    l = pl.program_id(0)
    i = pl.program_id(1)
    ni = pl.num_programs(1)

    @pl.when(jnp.logical_and(l == 0, i == 0))
    def _init():
        htc[...] = h0tc_ref[...]
        htv[...] = h0tv_ref[...]
        hcbf[...] = h0tc_ref[...].T.astype(_BF16)
        hvbf[...] = h0tv_ref[...].T.astype(_BF16)

    bt = b_ref[...]
    yc[pl.ds(i * BC, BC), :] = jax.lax.dot_general(
        bt, hvbf[...], (((1,), (0,)), ((), ())),
        preferred_element_type=_F32)
    part = jax.lax.dot_general(
        hcbf[pl.ds(i * BC, BC), :], bt, (((0,), (0,)), ((), ())),
        preferred_element_type=_F32)

    @pl.when(i == 0)
    def _set():
        ytv[...] = part

    @pl.when(i > 0)
    def _acc():
        ytv[...] += part

    @pl.when(i == ni - 1)
    def _epilogue():
        ytc = yc[...].T
        ytv_f = ytv[...]

        @pl.when(l == 0)
        def _deg():
            dgc[...] = ytc[1:2, :]
            dgv[...] = ytv_f[1:2, :]

        epsl = eps_ref[0]
        pc = ytc / dgc[...] + epsl * htc[...]
        pv = ytv_f / dgv[...] + epsl * htv[...]

        w1 = w1_ref[0]
        ac = jax.lax.dot_general(
            w1, pc.astype(_BF16), (((1,), (0,)), ((), ())),
            preferred_element_type=_F32)
        av = jax.lax.dot_general(
            w1, pv.astype(_BF16), (((1,), (0,)), ((), ())),
            preferred_element_type=_F32)
        ac = jnp.where(l == 0, w10_ref[...] * pc[0:1, :], ac) + b1_ref[0]
        av = jnp.where(l == 0, w10_ref[...] * pv[0:1, :], av) + b1_ref[0]
        ac, av = _bn_ref(ac, av, g1_ref[0], be1_ref[0])
        ac = jnp.maximum(ac, 0.0)
        av = jnp.maximum(av, 0.0)

        w2 = w2_ref[0]
        rc = jax.lax.dot_general(
            w2, ac.astype(_BF16), (((1,), (0,)), ((), ())),
            preferred_element_type=_F32) + b2_ref[0]
        rv = jax.lax.dot_general(
            w2, av.astype(_BF16), (((1,), (0,)), ((), ())),
            preferred_element_type=_F32) + b2_ref[0]
        rc, rv = _bn_ref(rc, rv, bng_ref[0], bnb_ref[0])
        hc_new = jnp.maximum(rc, 0.0)
        hv_new = jnp.maximum(rv, 0.0)

        htc[...] = hc_new
        htv[...] = hv_new
        hcbf[...] = hc_new.T.astype(_BF16)
        hvbf[...] = hv_new.T.astype(_BF16)
        out_ref[...] = hv_new


def _lfa_kernel(htv_ref, w_ref, lfab_ref, fc1wt_ref, fc1b_ref, out_ref):
    wq = w_ref[...].astype(_BF16)
    hv_cols = jnp.concatenate(
        [htv_ref[b].T.astype(_BF16) for b in range(BATCH)], axis=1)
    z_all = jax.lax.dot_general(
        wq, hv_cols, (((1,), (0,)), ((), ())),
        preferred_element_type=_F32)
    fc1wt = fc1wt_ref[...]
    for b in range(BATCH):
        z = z_all[:, HIDDEN * b:HIDDEN * (b + 1)] + lfab_ref[...]
        lg = jax.lax.dot_general(
            z.astype(_BF16), fc1wt, (((1,), (0,)), ((), ())),
            preferred_element_type=_F32) + fc1b_ref[...]
        mx = jnp.max(lg, axis=1, keepdims=True)
        e = jnp.exp(lg - mx)
        out_ref[b] = e / jnp.sum(e, axis=1, keepdims=True)


def kernel(batch_size, biggraph, clause_feat, var_feat, graph_pooler, params):
    del batch_size, graph_pooler
    nl = NUM_LAYERS - 1

    bb = biggraph.astype(_BF16)

    h0tc = jnp.zeros((HIDDEN, C), _F32)
    h0tc = h0tc.at[0, :].set(clause_feat[:, 0])
    h0tc = h0tc.at[1, :].set(1.0)
    h0tv = jnp.zeros((HIDDEN, V), _F32)
    h0tv = h0tv.at[0, :].set(var_feat[:, 0])
    h0tv = h0tv.at[1, :].set(1.0)

    w1s = []
    for lp in params['mlps']:
        w1 = lp['w1']
        if w1.shape[1] < HIDDEN:
            w1 = jnp.pad(w1, ((0, 0), (0, HIDDEN - w1.shape[1])))
        w1s.append(w1)
    w1s = jnp.stack(w1s).astype(_BF16)
    w2s = jnp.stack([lp['w2'] for lp in params['mlps']]).astype(_BF16)
    b1s = jnp.stack([lp['b1'] for lp in params['mlps']])[:, :, None]
    g1s = jnp.stack([lp['g1'] for lp in params['mlps']])[:, :, None]
    be1s = jnp.stack([lp['be1'] for lp in params['mlps']])[:, :, None]
    b2s = jnp.stack([lp['b2'] for lp in params['mlps']])[:, :, None]
    bngs = jnp.stack(list(params['bn_g']))[:, :, None]
    bnbs = jnp.stack(list(params['bn_b']))[:, :, None]
    epss = jnp.broadcast_to((1.0 + params['eps'])[:, None, None],
                            (nl, HIDDEN, 1))

    lspec = lambda blk: pl.BlockSpec(blk, lambda l, i: (l, 0, 0))
    htv_fin = pl.pallas_call(
        _gnn_kernel,
        grid=(nl, NI),
        in_specs=[
            pl.BlockSpec((BC, V), lambda l, i: (i, 0)),
            pl.BlockSpec((HIDDEN, C), lambda l, i: (0, 0)),
            pl.BlockSpec((HIDDEN, V), lambda l, i: (0, 0)),
            lspec((1, HIDDEN, HIDDEN)),
            lspec((1, HIDDEN, HIDDEN)),
            lspec((1, HIDDEN, 1)),
            lspec((1, HIDDEN, 1)),
            lspec((1, HIDDEN, 1)),
            lspec((1, HIDDEN, 1)),
            lspec((1, HIDDEN, 1)),
            lspec((1, HIDDEN, 1)),
            lspec((1, HIDDEN, 1)),
            pl.BlockSpec((HIDDEN, 1), lambda l, i: (0, 0)),
        ],
        out_specs=pl.BlockSpec((HIDDEN, V), lambda l, i: (0, 0)),
        out_shape=jax.ShapeDtypeStruct((HIDDEN, V), _F32),
        scratch_shapes=[
            pltpu.VMEM((HIDDEN, C), _F32),
            pltpu.VMEM((HIDDEN, V), _F32),
            pltpu.VMEM((C, HIDDEN), _BF16),
            pltpu.VMEM((V, HIDDEN), _BF16),
            pltpu.VMEM((C, HIDDEN), _F32),
            pltpu.VMEM((HIDDEN, V), _F32),
            pltpu.VMEM((1, C), _F32),
            pltpu.VMEM((1, V), _F32),
        ],
        compiler_params=pltpu.CompilerParams(
            dimension_semantics=("arbitrary", "arbitrary")),
    )(bb, h0tc, h0tv, w1s, w2s, b1s, g1s, be1s, b2s, bngs, bnbs, epss,
      params['mlps'][0]['w1'][:, 0:1])

    htv_b = htv_fin.reshape(HIDDEN, BATCH, MAXVAR).transpose(1, 0, 2)

    probs = pl.pallas_call(
        _lfa_kernel,
        out_shape=jax.ShapeDtypeStruct((BATCH, MAXVAR, OUT_DIM), _F32),
    )(htv_b, params['var_lfa_w'], params['var_lfa_b'][:, None],
      params['fc1_w'].T.astype(_BF16), params['fc1_b'][None, :])

    return probs.reshape(BATCH * MAXVAR, OUT_DIM)


# speedup vs baseline: 2.3274x; 1.0513x over previous
import jax
import jax.numpy as jnp
from jax.experimental import pallas as pl
from jax.experimental.pallas import tpu as pltpu

NUM_LAYERS = 10
HIDDEN = 16
OUT_DIM = 2
MAXCLAUSE = 1000
MAXVAR = 2000
BATCH = 4
BN_EPS = 1e-5

C = MAXCLAUSE * BATCH
V = MAXVAR * BATCH
N_TOT = 12000.0
BC0 = 400
NI0 = C // BC0
BC = 800
NI = C // BC

_F32 = jnp.float32
_BF16 = jnp.bfloat16


def _bn_ref(xc, xv, g, b):
    s = jnp.sum(xc, axis=1, keepdims=True) + jnp.sum(xv, axis=1, keepdims=True)
    m = s / N_TOT
    dc = xc - m
    dv = xv - m
    q = jnp.sum(dc * dc, axis=1, keepdims=True) + jnp.sum(dv * dv, axis=1, keepdims=True)
    v = q / N_TOT
    sd = jnp.sqrt(v + BN_EPS)
    return (dc / sd) * g + b, (dv / sd) * g + b


def _l0_kernel(b_ref, cft_ref, vft_ref, w10_ref, b1_ref, g1_ref, be1_ref,
               w2_ref, b2_ref, bng_ref, bnb_ref, eps_ref,
               bb_ref, htc1_ref, htv1_ref, hcbf1_ref, hvbf1_ref, dgc_ref, dgv_ref,
               htc0, htv0, hcbf0, hvbf0, yc, ytv):
    i = pl.program_id(0)
    ni = pl.num_programs(0)

    @pl.when(i == 0)
    def _init():
        zc = jnp.zeros((HIDDEN - 2, C), _F32)
        zv = jnp.zeros((HIDDEN - 2, V), _F32)
        h0tc = jnp.concatenate([cft_ref[...], jnp.ones((1, C), _F32), zc], axis=0)
        h0tv = jnp.concatenate([vft_ref[...], jnp.ones((1, V), _F32), zv], axis=0)
        htc0[...] = h0tc
        htv0[...] = h0tv
        hcbf0[...] = h0tc.T.astype(_BF16)
        hvbf0[...] = h0tv.T.astype(_BF16)

    bt = b_ref[...].astype(_BF16)
    bb_ref[...] = bt
    yc[pl.ds(i * BC0, BC0), :] = jax.lax.dot_general(
        bt, hvbf0[...], (((1,), (0,)), ((), ())),
        preferred_element_type=_F32)
    part = jax.lax.dot_general(
        hcbf0[pl.ds(i * BC0, BC0), :], bt, (((0,), (0,)), ((), ())),
        preferred_element_type=_F32)

    @pl.when(i == 0)
    def _set():
        ytv[...] = part

    @pl.when(i > 0)
    def _acc():
        ytv[...] += part

    @pl.when(i == ni - 1)
    def _epilogue():
        ytc = yc[...].T
        ytv_f = ytv[...]
        dgc = ytc[1:2, :]
        dgv = ytv_f[1:2, :]
        dgc_ref[...] = dgc
        dgv_ref[...] = dgv

        epsl = eps_ref[0]
        pc = ytc / dgc + epsl * htc0[...]
        pv = ytv_f / dgv + epsl * htv0[...]

        ac = w10_ref[...] * pc[0:1, :] + b1_ref[0]
        av = w10_ref[...] * pv[0:1, :] + b1_ref[0]
        ac, av = _bn_ref(ac, av, g1_ref[0], be1_ref[0])
        ac = jnp.maximum(ac, 0.0)
        av = jnp.maximum(av, 0.0)

        w2 = w2_ref[0]
        rc = jax.lax.dot_general(
            w2, ac.astype(_BF16), (((1,), (0,)), ((), ())),
            preferred_element_type=_F32) + b2_ref[0]
        rv = jax.lax.dot_general(
            w2, av.astype(_BF16), (((1,), (0,)), ((), ())),
            preferred_element_type=_F32) + b2_ref[0]
        rc, rv = _bn_ref(rc, rv, bng_ref[0], bnb_ref[0])
        hc_new = jnp.maximum(rc, 0.0)
        hv_new = jnp.maximum(rv, 0.0)

        htc1_ref[...] = hc_new
        htv1_ref[...] = hv_new
        hcbf1_ref[...] = hc_new.T.astype(_BF16)
        hvbf1_ref[...] = hv_new.T.astype(_BF16)


def _gnn_kernel(b_ref, htc1_ref, htv1_ref, hcbf1_ref, hvbf1_ref, dgc_ref,
                dgv_ref, w1_ref, w2_ref, b1_ref, g1_ref, be1_ref, b2_ref,
                bng_ref, bnb_ref, eps_ref, out_ref,
                htc, htv, hcbf, hvbf, yc, ytv):
    l = pl.program_id(0)
    i = pl.program_id(1)
    ni = pl.num_programs(1)

    @pl.when(jnp.logical_and(l == 0, i == 0))
    def _init():
        htc[...] = htc1_ref[...]
        htv[...] = htv1_ref[...]
        hcbf[...] = hcbf1_ref[...]
        hvbf[...] = hvbf1_ref[...]

    bt = b_ref[...]
    yc[pl.ds(i * BC, BC), :] = jax.lax.dot_general(
        bt, hvbf[...], (((1,), (0,)), ((), ())),
        preferred_element_type=_F32)
    part = jax.lax.dot_general(
        hcbf[pl.ds(i * BC, BC), :], bt, (((0,), (0,)), ((), ())),
        preferred_element_type=_F32)

    @pl.when(i == 0)
    def _set():
        ytv[...] = part

    @pl.when(i > 0)
    def _acc():
        ytv[...] += part

    @pl.when(i == ni - 1)
    def _epilogue():
        ytc = yc[...].T
        ytv_f = ytv[...]

        epsl = eps_ref[0]
        pc = ytc / dgc_ref[...] + epsl * htc[...]
        pv = ytv_f / dgv_ref[...] + epsl * htv[...]

        w1 = w1_ref[0]
        ac = jax.lax.dot_general(
            w1, pc.astype(_BF16), (((1,), (0,)), ((), ())),
            preferred_element_type=_F32) + b1_ref[0]
        av = jax.lax.dot_general(
            w1, pv.astype(_BF16), (((1,), (0,)), ((), ())),
            preferred_element_type=_F32) + b1_ref[0]
        ac, av = _bn_ref(ac, av, g1_ref[0], be1_ref[0])
        ac = jnp.maximum(ac, 0.0)
        av = jnp.maximum(av, 0.0)

        w2 = w2_ref[0]
        rc = jax.lax.dot_general(
            w2, ac.astype(_BF16), (((1,), (0,)), ((), ())),
            preferred_element_type=_F32) + b2_ref[0]
        rv = jax.lax.dot_general(
            w2, av.astype(_BF16), (((1,), (0,)), ((), ())),
            preferred_element_type=_F32) + b2_ref[0]
        rc, rv = _bn_ref(rc, rv, bng_ref[0], bnb_ref[0])
        hc_new = jnp.maximum(rc, 0.0)
        hv_new = jnp.maximum(rv, 0.0)

        htc[...] = hc_new
        htv[...] = hv_new
        hcbf[...] = hc_new.T.astype(_BF16)
        hvbf[...] = hv_new.T.astype(_BF16)
        out_ref[...] = hv_new


def _lfa_kernel(htv_ref, w_ref, lfab_ref, fc1wt_ref, fc1b_ref, out_ref):
    wq = w_ref[...].astype(_BF16)
    hv_cols = jnp.concatenate(
        [htv_ref[b].T.astype(_BF16) for b in range(BATCH)], axis=1)
    z_all = jax.lax.dot_general(
        wq, hv_cols, (((1,), (0,)), ((), ())),
        preferred_element_type=_F32)
    fc1wt = fc1wt_ref[...]
    for b in range(BATCH):
        z = z_all[:, HIDDEN * b:HIDDEN * (b + 1)] + lfab_ref[...]
        lg = jax.lax.dot_general(
            z.astype(_BF16), fc1wt, (((1,), (0,)), ((), ())),
            preferred_element_type=_F32) + fc1b_ref[...]
        mx = jnp.max(lg, axis=1, keepdims=True)
        e = jnp.exp(lg - mx)
        out_ref[b] = e / jnp.sum(e, axis=1, keepdims=True)


def kernel(batch_size, biggraph, clause_feat, var_feat, graph_pooler, params):
    del batch_size, graph_pooler
    nl = NUM_LAYERS - 1

    w1s = []
    for lp in params['mlps']:
        w1 = lp['w1']
        if w1.shape[1] < HIDDEN:
            w1 = jnp.pad(w1, ((0, 0), (0, HIDDEN - w1.shape[1])))
        w1s.append(w1)
    w1s = jnp.stack(w1s).astype(_BF16)
    w2s = jnp.stack([lp['w2'] for lp in params['mlps']]).astype(_BF16)
    b1s = jnp.stack([lp['b1'] for lp in params['mlps']])[:, :, None]
    g1s = jnp.stack([lp['g1'] for lp in params['mlps']])[:, :, None]
    be1s = jnp.stack([lp['be1'] for lp in params['mlps']])[:, :, None]
    b2s = jnp.stack([lp['b2'] for lp in params['mlps']])[:, :, None]
    bngs = jnp.stack(list(params['bn_g']))[:, :, None]
    bnbs = jnp.stack(list(params['bn_b']))[:, :, None]
    epss = jnp.broadcast_to((1.0 + params['eps'])[:, None, None],
                            (nl, HIDDEN, 1))

    cspec = lambda blk: pl.BlockSpec(blk, lambda i: (0, 0))
    bb, htc1, htv1, hcbf1, hvbf1, dgc, dgv = pl.pallas_call(
        _l0_kernel,
        grid=(NI0,),
        in_specs=[
            pl.BlockSpec((BC0, V), lambda i: (i, 0)),
            cspec((1, C)),
            cspec((1, V)),
            cspec((HIDDEN, 1)),
            pl.BlockSpec((1, HIDDEN, 1), lambda i: (0, 0, 0)),
            pl.BlockSpec((1, HIDDEN, 1), lambda i: (0, 0, 0)),
            pl.BlockSpec((1, HIDDEN, 1), lambda i: (0, 0, 0)),
            pl.BlockSpec((1, HIDDEN, HIDDEN), lambda i: (0, 0, 0)),
            pl.BlockSpec((1, HIDDEN, 1), lambda i: (0, 0, 0)),
            pl.BlockSpec((1, HIDDEN, 1), lambda i: (0, 0, 0)),
            pl.BlockSpec((1, HIDDEN, 1), lambda i: (0, 0, 0)),
            pl.BlockSpec((1, HIDDEN, 1), lambda i: (0, 0, 0)),
        ],
        out_specs=[
            pl.BlockSpec((BC0, V), lambda i: (i, 0)),
            cspec((HIDDEN, C)),
            cspec((HIDDEN, V)),
            cspec((C, HIDDEN)),
            cspec((V, HIDDEN)),
            cspec((1, C)),
            cspec((1, V)),
        ],
        out_shape=[
            jax.ShapeDtypeStruct((C, V), _BF16),
            jax.ShapeDtypeStruct((HIDDEN, C), _F32),
            jax.ShapeDtypeStruct((HIDDEN, V), _F32),
            jax.ShapeDtypeStruct((C, HIDDEN), _BF16),
            jax.ShapeDtypeStruct((V, HIDDEN), _BF16),
            jax.ShapeDtypeStruct((1, C), _F32),
            jax.ShapeDtypeStruct((1, V), _F32),
        ],
        scratch_shapes=[
            pltpu.VMEM((HIDDEN, C), _F32),
            pltpu.VMEM((HIDDEN, V), _F32),
            pltpu.VMEM((C, HIDDEN), _BF16),
            pltpu.VMEM((V, HIDDEN), _BF16),
            pltpu.VMEM((C, HIDDEN), _F32),
            pltpu.VMEM((HIDDEN, V), _F32),
        ],
        compiler_params=pltpu.CompilerParams(
            dimension_semantics=("arbitrary",)),
    )(biggraph, clause_feat.reshape(1, C), var_feat.reshape(1, V),
      params['mlps'][0]['w1'][:, 0:1], b1s, g1s, be1s,
      w2s, b2s, bngs, bnbs, epss)

    lspec = lambda blk: pl.BlockSpec(blk, lambda l, i: (l + 1, 0, 0))
    htv_fin = pl.pallas_call(
        _gnn_kernel,
        grid=(nl - 1, NI),
        in_specs=[
            pl.BlockSpec((BC, V), lambda l, i: (i, 0)),
            pl.BlockSpec((HIDDEN, C), lambda l, i: (0, 0)),
            pl.BlockSpec((HIDDEN, V), lambda l, i: (0, 0)),
            pl.BlockSpec((C, HIDDEN), lambda l, i: (0, 0)),
            pl.BlockSpec((V, HIDDEN), lambda l, i: (0, 0)),
            pl.BlockSpec((1, C), lambda l, i: (0, 0)),
            pl.BlockSpec((1, V), lambda l, i: (0, 0)),
            lspec((1, HIDDEN, HIDDEN)),
            lspec((1, HIDDEN, HIDDEN)),
            lspec((1, HIDDEN, 1)),
            lspec((1, HIDDEN, 1)),
            lspec((1, HIDDEN, 1)),
            lspec((1, HIDDEN, 1)),
            lspec((1, HIDDEN, 1)),
            lspec((1, HIDDEN, 1)),
            lspec((1, HIDDEN, 1)),
        ],
        out_specs=pl.BlockSpec((HIDDEN, V), lambda l, i: (0, 0)),
        out_shape=jax.ShapeDtypeStruct((HIDDEN, V), _F32),
        scratch_shapes=[
            pltpu.VMEM((HIDDEN, C), _F32),
            pltpu.VMEM((HIDDEN, V), _F32),
            pltpu.VMEM((C, HIDDEN), _BF16),
            pltpu.VMEM((V, HIDDEN), _BF16),
            pltpu.VMEM((C, HIDDEN), _F32),
            pltpu.VMEM((HIDDEN, V), _F32),
        ],
        compiler_params=pltpu.CompilerParams(
            dimension_semantics=("arbitrary", "arbitrary")),
    )(bb, htc1, htv1, hcbf1, hvbf1, dgc, dgv,
      w1s, w2s, b1s, g1s, be1s, b2s, bngs, bnbs, epss)

    htv_b = htv_fin.reshape(HIDDEN, BATCH, MAXVAR).transpose(1, 0, 2)

    probs = pl.pallas_call(
        _lfa_kernel,
        out_shape=jax.ShapeDtypeStruct((BATCH, MAXVAR, OUT_DIM), _F32),
    )(htv_b, params['var_lfa_w'], params['var_lfa_b'][:, None],
      params['fc1_w'].T.astype(_BF16), params['fc1_b'][None, :])

    return probs.reshape(BATCH * MAXVAR, OUT_DIM)
